# baseline (speedup 1.0000x reference)
import dataclasses
import functools

import jax
import jax.numpy as jnp
from jax import lax
from jax.experimental import pallas as pl
from jax.experimental.pallas import tpu as pltpu
from jax.experimental.pallas import tpu_sc as plsc

N = 10000
E = 320000
D = 128
R = 8

NC = 2
NS = 16
RL = R // NC
EPC = E // NS
CHL = 96
MAXCH = -(-EPC // CHL) + (RL - 1)
LV = MAXCH * CHL
NPAD = 10112
CROWS = RL * NPAD
CROWS_PT = CROWS // NS
ACC_PT = NPAD // NS
NZC = 6
ACC_TAIL = ACC_PT - NZC * CHL
PACK = 16384

_mesh = plsc.VectorSubcoreMesh(core_axis_name="c", subcore_axis_name="s")

_sc_params = pltpu.CompilerParams()
for _f, _v in (("needs_layout_passes", False), ("use_tc_tiling_on_sc", False)):
  if _f in pltpu.CompilerParams.__dataclass_fields__:
    _sc_params = dataclasses.replace(_sc_params, **{_f: _v})


def _prep_body(pk_hbm,
               vlist_hbm, meta_hbm, counts_hbm,
               counts_sp, pbuf, vlist_v, meta_vv, ones_v, irow, czero):
  c = lax.axis_index("c")
  s = lax.axis_index("s")

  zf16 = jnp.zeros((16,), jnp.float32)
  of16 = jnp.ones((16,), jnp.float32)
  zi16 = jnp.zeros((16,), jnp.int32)

  @pl.loop(0, 632)
  def _(i):
    czero[i, :] = zf16

  @pl.loop(0, CHL)
  def _(i):
    ones_v[i, :] = of16

  @pl.loop(0, LV // 16)
  def _(i):
    vlist_v[pl.ds(i * 16, 16)] = zi16 + jnp.int32(N * 8)

  for j in range(4):
    pltpu.sync_copy(czero, counts_sp.at[pl.ds(s * CROWS_PT + j * 632, 632)])

  pltpu.sync_copy(pk_hbm.at[pl.ds(s * EPC, EPC)], pbuf)

  plsc.subcore_barrier()

  off = jnp.int32(0)
  ks = []
  starts = []
  idx16 = lax.iota(jnp.int32, 16)
  mvec = jnp.zeros((16,), jnp.int32)
  for rl in range(RL):
    r = c * RL + rl
    start = lax.div(off, jnp.int32(CHL))

    def step(i, off, r=r):
      v = pbuf[pl.ds(i * 16, 16)]
      m = jnp.bitwise_and(v, jnp.int32(7)) == r
      plsc.store_compressed(vlist_v.at[pl.ds(off, 16)], v, mask=m)
      return off + plsc.all_reduce_population_count(m)[0]

    end = lax.fori_loop(0, EPC // 16, step, off)
    k = end - start * jnp.int32(CHL)
    mvec = jnp.where(idx16 == rl, k, mvec)
    mvec = jnp.where(idx16 == RL + rl, start, mvec)
    ks.append(k)
    starts.append(start)
    off = lax.div(end + jnp.int32(CHL - 1), jnp.int32(CHL)) * jnp.int32(CHL)

  meta_vv[...] = mvec
  pltpu.sync_copy(vlist_v, vlist_hbm.at[c, s])
  pltpu.sync_copy(meta_vv, meta_hbm.at[c, s])

  for rl in range(RL):
    nch = lax.div(ks[rl] + jnp.int32(CHL - 1), jnp.int32(CHL))
    roff = jnp.int32(rl * NPAD)

    @pl.loop(0, nch)
    def _(j, start=starts[rl], roff=roff):
      base = (start + j) * CHL
      for q in range(CHL // 16):
        v = vlist_v[pl.ds(base + q * 16, 16)]
        dv = jnp.bitwise_and(lax.shift_right_logical(v, jnp.int32(3)),
                             jnp.int32(PACK - 1))
        irow[0, pl.ds(q * 16, 16)] = dv + roff
      pltpu.sync_copy(ones_v, counts_sp.at[irow.at[0]], add=True)

  plsc.subcore_barrier()

  for j in range(4):
    pltpu.sync_copy(counts_sp.at[pl.ds(s * CROWS_PT + j * 632, 632)], czero)
    pltpu.sync_copy(
        czero, counts_hbm.at[pl.ds(c * CROWS + s * CROWS_PT + j * 632, 632)])


_prep = pl.kernel(
    _prep_body,
    out_type=(
        jax.ShapeDtypeStruct((NC, NS, LV), jnp.int32),
        jax.ShapeDtypeStruct((NC, NS, 16), jnp.int32),
        jax.ShapeDtypeStruct((NC * CROWS, 16), jnp.float32),
    ),
    mesh=_mesh,
    scratch_types=[
        pltpu.VMEM_SHARED((CROWS, 16), jnp.float32),
        pltpu.VMEM((EPC,), jnp.int32),
        pltpu.VMEM((LV,), jnp.int32),
        pltpu.VMEM((16,), jnp.int32),
        pltpu.VMEM((CHL, 16), jnp.float32),
        pltpu.VMEM((1, CHL), jnp.int32),
        pltpu.VMEM((632, 16), jnp.float32),
    ],
    compiler_params=_sc_params,
)


def _agg_body(feats_hbm, vlist_hbm, meta_hbm, s_hbm,
              acc_sp, vbuf2, meta_vv, gbuf2, srow2, drow2,
              semL0, semL1, semG0, semG1, semS0, semS1, semZ, semW0, semW1):
  c = lax.axis_index("c")
  s = lax.axis_index("s")

  zb16 = jnp.zeros((32,), jnp.bfloat16)

  pltpu.sync_copy(meta_hbm.at[c, s], meta_vv)

  def unpack(p):
    for q in range(CHL // 16):
      v = vbuf2[p, pl.ds(q * 16, 16)]
      srow2[p, pl.ds(q * 16, 16)] = lax.shift_right_logical(v, jnp.int32(17))
      drow2[p, pl.ds(q * 16, 16)] = jnp.bitwise_and(
          lax.shift_right_logical(v, jnp.int32(3)), jnp.int32(PACK - 1))

  def fetch_list(chunk, p, sem):
    sems = (semL0, semL1)
    pltpu.async_copy(vlist_hbm.at[c, s, pl.ds(chunk * CHL, CHL)],
                     vbuf2.at[p], sems[p])

  for rl in range(RL):
    @pl.loop(0, CHL)
    def _(i):
      for q in range(D // 32):
        gbuf2[0, i, pl.ds(q * 32, 32)] = zb16

    zb = s * ACC_PT
    for t in range(NZC):
      pltpu.async_copy(gbuf2.at[0], acc_sp.at[pl.ds(zb + t * CHL, CHL)], semZ)
    pltpu.async_copy(gbuf2.at[0, pl.ds(0, ACC_TAIL)],
                     acc_sp.at[pl.ds(zb + NZC * CHL, ACC_TAIL)], semZ)
    for t in range(NZC):
      pltpu.make_async_copy(
          gbuf2.at[0], acc_sp.at[pl.ds(zb + t * CHL, CHL)], semZ).wait()
    pltpu.make_async_copy(gbuf2.at[0, pl.ds(0, ACC_TAIL)],
                          acc_sp.at[pl.ds(zb + NZC * CHL, ACC_TAIL)],
                          semZ).wait()
    plsc.subcore_barrier()

    mv = meta_vv[...]
    k = mv[rl]
    start = mv[RL + rl]
    nch = lax.div(k + jnp.int32(CHL - 1), jnp.int32(CHL))
    npair = lax.div(nch + jnp.int32(1), jnp.int32(2))

    @pl.when(nch > 0)
    def _():
      fetch_list(start, 0, semL0)

    @pl.loop(0, npair)
    def _(i):
      a = 2 * i
      b = 2 * i + 1

      @pl.when(b < nch)
      def _():
        fetch_list(start + b, 1, semL1)

      @pl.when(i > 0)
      def _():
        pltpu.make_async_copy(gbuf2.at[0], acc_sp.at[drow2.at[0]],
                              semS0).wait()
      pltpu.make_async_copy(vlist_hbm.at[c, s, pl.ds((start + a) * CHL, CHL)],
                            vbuf2.at[0], semL0).wait()
      unpack(0)
      pltpu.async_copy(feats_hbm.at[srow2.at[0]], gbuf2.at[0], semG0)

      @pl.when(b < nch)
      def _():
        @pl.when(a + 2 < nch)
        def _():
          fetch_list(start + a + 2, 0, semL0)

        @pl.when(i > 0)
        def _():
          pltpu.make_async_copy(gbuf2.at[1], acc_sp.at[drow2.at[1]],
                                semS1).wait()
        pltpu.make_async_copy(
            vlist_hbm.at[c, s, pl.ds((start + b) * CHL, CHL)],
            vbuf2.at[1], semL1).wait()
        unpack(1)
        pltpu.async_copy(feats_hbm.at[srow2.at[1]], gbuf2.at[1], semG1)

      pltpu.make_async_copy(feats_hbm.at[srow2.at[0]], gbuf2.at[0],
                            semG0).wait()
      pltpu.async_copy(gbuf2.at[0], acc_sp.at[drow2.at[0]], semS0, add=True)

      @pl.when(b < nch)
      def _():
        pltpu.make_async_copy(feats_hbm.at[srow2.at[1]], gbuf2.at[1],
                              semG1).wait()
        pltpu.async_copy(gbuf2.at[1], acc_sp.at[drow2.at[1]], semS1, add=True)

    @pl.when(nch > 0)
    def _():
      pltpu.make_async_copy(gbuf2.at[0], acc_sp.at[drow2.at[0]], semS0).wait()

    @pl.when(nch > 1)
    def _():
      pltpu.make_async_copy(gbuf2.at[1], acc_sp.at[drow2.at[1]], semS1).wait()

    plsc.subcore_barrier()

    r = c * RL + rl
    semW = (semW0, semW1)
    sizes = [CHL] * NZC + [ACC_TAIL]
    for t in range(NZC + 1):
      p = t % 2
      off = zb + t * CHL
      sz = sizes[t]
      if t >= 2:
        pltpu.make_async_copy(
            gbuf2.at[p, pl.ds(0, sizes[t - 2])],
            s_hbm.at[r, pl.ds(zb + (t - 2) * CHL, sizes[t - 2])],
            semW[p]).wait()
      pltpu.sync_copy(acc_sp.at[pl.ds(off, sz)], gbuf2.at[p, pl.ds(0, sz)])
      pltpu.async_copy(gbuf2.at[p, pl.ds(0, sz)],
                       s_hbm.at[r, pl.ds(off, sz)], semW[p])
    pltpu.make_async_copy(gbuf2.at[1, pl.ds(0, sizes[NZC - 1])],
                          s_hbm.at[r, pl.ds(zb + (NZC - 1) * CHL,
                                            sizes[NZC - 1])],
                          semW1).wait()
    pltpu.make_async_copy(gbuf2.at[0, pl.ds(0, ACC_TAIL)],
                          s_hbm.at[r, pl.ds(zb + NZC * CHL, ACC_TAIL)],
                          semW0).wait()


_agg = pl.kernel(
    _agg_body,
    out_type=jax.ShapeDtypeStruct((R, NPAD, D), jnp.bfloat16),
    mesh=_mesh,
    scratch_types=[
        pltpu.VMEM_SHARED((NPAD, D), jnp.bfloat16),
        pltpu.VMEM((2, CHL), jnp.int32),
        pltpu.VMEM((16,), jnp.int32),
        pltpu.VMEM((2, CHL, D), jnp.bfloat16),
        pltpu.VMEM((2, CHL), jnp.int32),
        pltpu.VMEM((2, CHL), jnp.int32),
        pltpu.SemaphoreType.DMA,
        pltpu.SemaphoreType.DMA,
        pltpu.SemaphoreType.DMA,
        pltpu.SemaphoreType.DMA,
        pltpu.SemaphoreType.DMA,
        pltpu.SemaphoreType.DMA,
        pltpu.SemaphoreType.DMA,
        pltpu.SemaphoreType.DMA,
        pltpu.SemaphoreType.DMA,
    ],
    compiler_params=_sc_params,
)


BN = 400


def _tc_root_body(x_ref, root_ref, b_ref, o_ref):
  o_ref[...] = jnp.dot(x_ref[...].astype(jnp.bfloat16), root_ref[...],
                       preferred_element_type=jnp.float32) + b_ref[...]


def _tc_root(feats, root, b):
  return pl.pallas_call(
      _tc_root_body,
      grid=(N // BN,),
      in_specs=[
          pl.BlockSpec((BN, D), lambda i: (i, 0)),
          pl.BlockSpec((D, D), lambda i: (0, 0)),
          pl.BlockSpec((1, D), lambda i: (0, 0)),
      ],
      out_specs=pl.BlockSpec((BN, D), lambda i: (i, 0)),
      out_shape=jax.ShapeDtypeStruct((N, D), jnp.float32),
  )(feats, root, b)


def _tc_layer_body(base_ref, s_ref, cnt_ref, w_ref, o_ref, ob_ref, *, relu):
  acc = base_ref[...]
  for r in range(R):
    cnt = cnt_ref[r][:, 0:1]
    norm = (1.0 / jnp.maximum(cnt, 1.0)).astype(jnp.bfloat16)
    acc = acc + jnp.dot(s_ref[r] * norm, w_ref[r],
                        preferred_element_type=jnp.float32)
  if relu:
    acc = jnp.maximum(acc, 0.0)
  o_ref[...] = acc
  ob_ref[...] = acc.astype(jnp.bfloat16)


def _tc_layer(base, S, counts, W, relu):
  body = functools.partial(_tc_layer_body, relu=relu)
  return pl.pallas_call(
      body,
      grid=(N // BN,),
      in_specs=[
          pl.BlockSpec((BN, D), lambda i: (i, 0)),
          pl.BlockSpec((R, BN, D), lambda i: (0, i, 0)),
          pl.BlockSpec((R, BN, 16), lambda i: (0, i, 0)),
          pl.BlockSpec((R, D, D), lambda i: (0, 0, 0)),
      ],
      out_specs=[pl.BlockSpec((BN, D), lambda i: (i, 0)),
                 pl.BlockSpec((BN, D), lambda i: (i, 0))],
      out_shape=(jax.ShapeDtypeStruct((N, D), jnp.float32),
                 jax.ShapeDtypeStruct((N, D), jnp.bfloat16)),
  )(base, S, counts, W)


def kernel(x, edge_index, edge_type, W1, root1, b1, W2, root2, b2):
  src = edge_index[0]
  dst = edge_index[1]
  packed = (src * PACK + dst) * 8 + edge_type
  vlist, meta, counts = _prep(packed)
  cnts = counts.reshape(R, NPAD, 16)
  w1b = W1.astype(jnp.bfloat16)
  w2b = W2.astype(jnp.bfloat16)
  r1b = root1.astype(jnp.bfloat16)
  r2b = root2.astype(jnp.bfloat16)
  base1 = _tc_root(x, r1b, b1.reshape(1, D))
  S1 = _agg(x.astype(jnp.bfloat16), vlist, meta)
  h, hb = _tc_layer(base1, S1, cnts, w1b, True)
  base2 = _tc_root(h, r2b, b2.reshape(1, D))
  S2 = _agg(hb, vlist, meta)
  out, _ = _tc_layer(base2, S2, cnts, w2b, False)
  return out


# speedup vs baseline: 1.0201x; 1.0201x over previous
import dataclasses
import functools

import jax
import jax.numpy as jnp
from jax import lax
from jax.experimental import pallas as pl
from jax.experimental.pallas import tpu as pltpu
from jax.experimental.pallas import tpu_sc as plsc

N = 10000
E = 320000
D = 128
R = 8

NC = 2
NS = 16
RL = R // NC
EPC = E // NS
CHL = 96
MAXCH = -(-EPC // CHL) + (RL - 1)
LV = MAXCH * CHL
NPAD = 10112
CROWS = RL * NPAD
CROWS_PT = CROWS // NS
ACC_PT = NPAD // NS
NZC = 6
ACC_TAIL = ACC_PT - NZC * CHL
PACK = 16384

_mesh = plsc.VectorSubcoreMesh(core_axis_name="c", subcore_axis_name="s")

_sc_params = pltpu.CompilerParams()
for _f, _v in (("needs_layout_passes", False), ("use_tc_tiling_on_sc", False)):
  if _f in pltpu.CompilerParams.__dataclass_fields__:
    _sc_params = dataclasses.replace(_sc_params, **{_f: _v})


def _prep_body(pk_hbm,
               vlist_hbm, meta_hbm, counts_hbm,
               counts_sp, pbuf, vlist_v, meta_vv, ones_v, irow, czero):
  c = lax.axis_index("c")
  s = lax.axis_index("s")

  zf16 = jnp.zeros((16,), jnp.float32)
  of16 = jnp.ones((16,), jnp.float32)
  zi16 = jnp.zeros((16,), jnp.int32)

  @pl.loop(0, 632)
  def _(i):
    czero[i, :] = zf16

  @pl.loop(0, CHL)
  def _(i):
    ones_v[i, :] = of16

  @pl.loop(0, LV // 16)
  def _(i):
    vlist_v[pl.ds(i * 16, 16)] = zi16 + jnp.int32(N * 8)

  for j in range(4):
    pltpu.sync_copy(czero, counts_sp.at[pl.ds(s * CROWS_PT + j * 632, 632)])

  pltpu.sync_copy(pk_hbm.at[pl.ds(s * EPC, EPC)], pbuf)

  plsc.subcore_barrier()

  off = jnp.int32(0)
  ks = []
  starts = []
  idx16 = lax.iota(jnp.int32, 16)
  mvec = jnp.zeros((16,), jnp.int32)
  for rl in range(RL):
    r = c * RL + rl
    start = lax.div(off, jnp.int32(CHL))

    def step(i, off, r=r):
      v = pbuf[pl.ds(i * 16, 16)]
      m = jnp.bitwise_and(v, jnp.int32(7)) == r
      plsc.store_compressed(vlist_v.at[pl.ds(off, 16)], v, mask=m)
      return off + jnp.max(plsc.all_reduce_population_count(m))

    end = lax.fori_loop(0, EPC // 16, step, off)
    k = end - start * jnp.int32(CHL)
    mvec = jnp.where(idx16 == rl, k, mvec)
    mvec = jnp.where(idx16 == RL + rl, start, mvec)
    ks.append(k)
    starts.append(start)
    off = lax.div(end + jnp.int32(CHL - 1), jnp.int32(CHL)) * jnp.int32(CHL)

  meta_vv[...] = mvec
  pltpu.sync_copy(vlist_v, vlist_hbm.at[c, s])
  pltpu.sync_copy(meta_vv, meta_hbm.at[c, s])

  for rl in range(RL):
    nch = lax.div(ks[rl] + jnp.int32(CHL - 1), jnp.int32(CHL))
    roff = jnp.int32(rl * NPAD)

    @pl.loop(0, nch)
    def _(j, start=starts[rl], roff=roff):
      base = (start + j) * CHL
      for q in range(CHL // 16):
        v = vlist_v[pl.ds(base + q * 16, 16)]
        dv = jnp.bitwise_and(lax.shift_right_logical(v, jnp.int32(3)),
                             jnp.int32(PACK - 1))
        irow[0, pl.ds(q * 16, 16)] = dv + roff
      pltpu.sync_copy(ones_v, counts_sp.at[irow.at[0]], add=True)

  plsc.subcore_barrier()

  for j in range(4):
    pltpu.sync_copy(counts_sp.at[pl.ds(s * CROWS_PT + j * 632, 632)], czero)
    pltpu.sync_copy(
        czero, counts_hbm.at[pl.ds(c * CROWS + s * CROWS_PT + j * 632, 632)])


_prep = pl.kernel(
    _prep_body,
    out_type=(
        jax.ShapeDtypeStruct((NC, NS, LV), jnp.int32),
        jax.ShapeDtypeStruct((NC, NS, 16), jnp.int32),
        jax.ShapeDtypeStruct((NC * CROWS, 16), jnp.float32),
    ),
    mesh=_mesh,
    scratch_types=[
        pltpu.VMEM_SHARED((CROWS, 16), jnp.float32),
        pltpu.VMEM((EPC,), jnp.int32),
        pltpu.VMEM((LV,), jnp.int32),
        pltpu.VMEM((16,), jnp.int32),
        pltpu.VMEM((CHL, 16), jnp.float32),
        pltpu.VMEM((1, CHL), jnp.int32),
        pltpu.VMEM((632, 16), jnp.float32),
    ],
    compiler_params=_sc_params,
)


def _agg_body(feats_hbm, vlist_hbm, meta_hbm, s_hbm,
              acc_sp, vbuf2, meta_vv, gbuf2, srow2, drow2,
              semL0, semL1, semG0, semG1, semS0, semS1, semZ, semW0, semW1):
  c = lax.axis_index("c")
  s = lax.axis_index("s")

  zb16 = jnp.zeros((32,), jnp.bfloat16)

  pltpu.sync_copy(meta_hbm.at[c, s], meta_vv)

  def unpack(p):
    for q in range(CHL // 16):
      v = vbuf2[p, pl.ds(q * 16, 16)]
      srow2[p, pl.ds(q * 16, 16)] = lax.shift_right_logical(v, jnp.int32(17))
      drow2[p, pl.ds(q * 16, 16)] = jnp.bitwise_and(
          lax.shift_right_logical(v, jnp.int32(3)), jnp.int32(PACK - 1))

  def fetch_list(chunk, p, sem):
    sems = (semL0, semL1)
    pltpu.async_copy(vlist_hbm.at[c, s, pl.ds(chunk * CHL, CHL)],
                     vbuf2.at[p], sems[p])

  for rl in range(RL):
    @pl.loop(0, CHL)
    def _(i):
      for q in range(D // 32):
        gbuf2[0, i, pl.ds(q * 32, 32)] = zb16

    zb = s * ACC_PT
    for t in range(NZC):
      pltpu.async_copy(gbuf2.at[0], acc_sp.at[pl.ds(zb + t * CHL, CHL)], semZ)
    pltpu.async_copy(gbuf2.at[0, pl.ds(0, ACC_TAIL)],
                     acc_sp.at[pl.ds(zb + NZC * CHL, ACC_TAIL)], semZ)
    for t in range(NZC):
      pltpu.make_async_copy(
          gbuf2.at[0], acc_sp.at[pl.ds(zb + t * CHL, CHL)], semZ).wait()
    pltpu.make_async_copy(gbuf2.at[0, pl.ds(0, ACC_TAIL)],
                          acc_sp.at[pl.ds(zb + NZC * CHL, ACC_TAIL)],
                          semZ).wait()
    plsc.subcore_barrier()

    mv = meta_vv[...]
    k = mv[rl]
    start = mv[RL + rl]
    nch = lax.div(k + jnp.int32(CHL - 1), jnp.int32(CHL))
    npair = lax.div(nch + jnp.int32(1), jnp.int32(2))

    @pl.when(nch > 0)
    def _():
      fetch_list(start, 0, semL0)

    @pl.loop(0, npair)
    def _(i):
      a = 2 * i
      b = 2 * i + 1

      @pl.when(b < nch)
      def _():
        fetch_list(start + b, 1, semL1)

      @pl.when(i > 0)
      def _():
        pltpu.make_async_copy(gbuf2.at[0], acc_sp.at[drow2.at[0]],
                              semS0).wait()
      pltpu.make_async_copy(vlist_hbm.at[c, s, pl.ds((start + a) * CHL, CHL)],
                            vbuf2.at[0], semL0).wait()
      unpack(0)
      pltpu.async_copy(feats_hbm.at[srow2.at[0]], gbuf2.at[0], semG0)

      @pl.when(b < nch)
      def _():
        @pl.when(a + 2 < nch)
        def _():
          fetch_list(start + a + 2, 0, semL0)

        @pl.when(i > 0)
        def _():
          pltpu.make_async_copy(gbuf2.at[1], acc_sp.at[drow2.at[1]],
                                semS1).wait()
        pltpu.make_async_copy(
            vlist_hbm.at[c, s, pl.ds((start + b) * CHL, CHL)],
            vbuf2.at[1], semL1).wait()
        unpack(1)
        pltpu.async_copy(feats_hbm.at[srow2.at[1]], gbuf2.at[1], semG1)

      pltpu.make_async_copy(feats_hbm.at[srow2.at[0]], gbuf2.at[0],
                            semG0).wait()
      pltpu.async_copy(gbuf2.at[0], acc_sp.at[drow2.at[0]], semS0, add=True)

      @pl.when(b < nch)
      def _():
        pltpu.make_async_copy(feats_hbm.at[srow2.at[1]], gbuf2.at[1],
                              semG1).wait()
        pltpu.async_copy(gbuf2.at[1], acc_sp.at[drow2.at[1]], semS1, add=True)

    @pl.when(nch > 0)
    def _():
      pltpu.make_async_copy(gbuf2.at[0], acc_sp.at[drow2.at[0]], semS0).wait()

    @pl.when(nch > 1)
    def _():
      pltpu.make_async_copy(gbuf2.at[1], acc_sp.at[drow2.at[1]], semS1).wait()

    plsc.subcore_barrier()

    r = c * RL + rl
    semW = (semW0, semW1)
    sizes = [CHL] * NZC + [ACC_TAIL]
    for t in range(NZC + 1):
      p = t % 2
      off = zb + t * CHL
      sz = sizes[t]
      if t >= 2:
        pltpu.make_async_copy(
            gbuf2.at[p, pl.ds(0, sizes[t - 2])],
            s_hbm.at[r, pl.ds(zb + (t - 2) * CHL, sizes[t - 2])],
            semW[p]).wait()
      pltpu.sync_copy(acc_sp.at[pl.ds(off, sz)], gbuf2.at[p, pl.ds(0, sz)])
      pltpu.async_copy(gbuf2.at[p, pl.ds(0, sz)],
                       s_hbm.at[r, pl.ds(off, sz)], semW[p])
    pltpu.make_async_copy(gbuf2.at[1, pl.ds(0, sizes[NZC - 1])],
                          s_hbm.at[r, pl.ds(zb + (NZC - 1) * CHL,
                                            sizes[NZC - 1])],
                          semW1).wait()
    pltpu.make_async_copy(gbuf2.at[0, pl.ds(0, ACC_TAIL)],
                          s_hbm.at[r, pl.ds(zb + NZC * CHL, ACC_TAIL)],
                          semW0).wait()


_agg = pl.kernel(
    _agg_body,
    out_type=jax.ShapeDtypeStruct((R, NPAD, D), jnp.bfloat16),
    mesh=_mesh,
    scratch_types=[
        pltpu.VMEM_SHARED((NPAD, D), jnp.bfloat16),
        pltpu.VMEM((2, CHL), jnp.int32),
        pltpu.VMEM((16,), jnp.int32),
        pltpu.VMEM((2, CHL, D), jnp.bfloat16),
        pltpu.VMEM((2, CHL), jnp.int32),
        pltpu.VMEM((2, CHL), jnp.int32),
        pltpu.SemaphoreType.DMA,
        pltpu.SemaphoreType.DMA,
        pltpu.SemaphoreType.DMA,
        pltpu.SemaphoreType.DMA,
        pltpu.SemaphoreType.DMA,
        pltpu.SemaphoreType.DMA,
        pltpu.SemaphoreType.DMA,
        pltpu.SemaphoreType.DMA,
        pltpu.SemaphoreType.DMA,
    ],
    compiler_params=_sc_params,
)


BN = 400


def _tc_layer_body(x_ref, s_ref, cnt_ref, w_ref, root_ref, b_ref,
                   o_ref, ob_ref, *, relu):
  acc = jnp.dot(x_ref[...].astype(jnp.bfloat16), root_ref[...],
                preferred_element_type=jnp.float32)
  acc = acc + b_ref[...]
  for r in range(R):
    cnt = cnt_ref[r][:, 0:1]
    norm = (1.0 / jnp.maximum(cnt, 1.0)).astype(jnp.bfloat16)
    acc = acc + jnp.dot(s_ref[r] * norm, w_ref[r],
                        preferred_element_type=jnp.float32)
  if relu:
    acc = jnp.maximum(acc, 0.0)
  o_ref[...] = acc
  ob_ref[...] = acc.astype(jnp.bfloat16)


def _tc_layer(feats, S, counts, W, root, b, relu):
  body = functools.partial(_tc_layer_body, relu=relu)
  return pl.pallas_call(
      body,
      grid=(N // BN,),
      in_specs=[
          pl.BlockSpec((BN, D), lambda i: (i, 0)),
          pl.BlockSpec((R, BN, D), lambda i: (0, i, 0)),
          pl.BlockSpec((R, BN, 16), lambda i: (0, i, 0)),
          pl.BlockSpec((R, D, D), lambda i: (0, 0, 0)),
          pl.BlockSpec((D, D), lambda i: (0, 0)),
          pl.BlockSpec((1, D), lambda i: (0, 0)),
      ],
      out_specs=[pl.BlockSpec((BN, D), lambda i: (i, 0)),
                 pl.BlockSpec((BN, D), lambda i: (i, 0))],
      out_shape=(jax.ShapeDtypeStruct((N, D), jnp.float32),
                 jax.ShapeDtypeStruct((N, D), jnp.bfloat16)),
  )(feats, S, counts, W, root, b)


def kernel(x, edge_index, edge_type, W1, root1, b1, W2, root2, b2):
  src = edge_index[0]
  dst = edge_index[1]
  packed = (src * PACK + dst) * 8 + edge_type
  vlist, meta, counts = _prep(packed)
  cnts = counts.reshape(R, NPAD, 16)
  w1b = W1.astype(jnp.bfloat16)
  w2b = W2.astype(jnp.bfloat16)
  r1b = root1.astype(jnp.bfloat16)
  r2b = root2.astype(jnp.bfloat16)
  S1 = _agg(x.astype(jnp.bfloat16), vlist, meta)
  h, hb = _tc_layer(x, S1, cnts, w1b, r1b, b1.reshape(1, D), True)
  S2 = _agg(hb, vlist, meta)
  out, _ = _tc_layer(h, S2, cnts, w2b, r2b, b2.reshape(1, D), False)
  return out


# speedup vs baseline: 1.1264x; 1.1042x over previous
import dataclasses
import functools

import jax
import jax.numpy as jnp
from jax import lax
from jax.experimental import pallas as pl
from jax.experimental.pallas import tpu as pltpu
from jax.experimental.pallas import tpu_sc as plsc

N = 10000
E = 320000
D = 128
R = 8

NC = 2
NS = 16
RL = R // NC
EPC = E // NS
CHL = 96
MAXCH = -(-EPC // CHL) + (RL - 1)
LV = MAXCH * CHL
NPAD = 10112
CROWS = RL * NPAD
CROWS_PT = CROWS // NS
ACC_PT = NPAD // NS
NZC = 6
ACC_TAIL = ACC_PT - NZC * CHL
PACK = 16384

_mesh = plsc.VectorSubcoreMesh(core_axis_name="c", subcore_axis_name="s")

_sc_params = pltpu.CompilerParams()
for _f, _v in (("needs_layout_passes", False), ("use_tc_tiling_on_sc", False)):
  if _f in pltpu.CompilerParams.__dataclass_fields__:
    _sc_params = dataclasses.replace(_sc_params, **{_f: _v})


def _prep_body(pk_hbm,
               vlist_hbm, meta_hbm, counts_hbm,
               counts_sp, pbuf, vlist_v, meta_vv, ones_v, irow, czero):
  c = lax.axis_index("c")
  s = lax.axis_index("s")

  zf16 = jnp.zeros((16,), jnp.float32)
  of16 = jnp.ones((16,), jnp.float32)
  zi16 = jnp.zeros((16,), jnp.int32)

  @pl.loop(0, 632)
  def _(i):
    czero[i, :] = zf16

  @pl.loop(0, CHL)
  def _(i):
    ones_v[i, :] = of16

  @pl.loop(0, LV // 16)
  def _(i):
    vlist_v[pl.ds(i * 16, 16)] = zi16 + jnp.int32(N * 8)

  for j in range(4):
    pltpu.sync_copy(czero, counts_sp.at[pl.ds(s * CROWS_PT + j * 632, 632)])

  pltpu.sync_copy(pk_hbm.at[pl.ds(s * EPC, EPC)], pbuf)

  plsc.subcore_barrier()

  off = jnp.int32(0)
  ks = []
  starts = []
  idx16 = lax.iota(jnp.int32, 16)
  mvec = jnp.zeros((16,), jnp.int32)
  for rl in range(RL):
    r = c * RL + rl
    start = lax.div(off, jnp.int32(CHL))

    def step(i, off, r=r):
      v = pbuf[pl.ds(i * 16, 16)]
      m = jnp.bitwise_and(v, jnp.int32(7)) == r
      plsc.store_compressed(vlist_v.at[pl.ds(off, 16)], v, mask=m)
      return off + jnp.max(plsc.all_reduce_population_count(m))

    end = lax.fori_loop(0, EPC // 16, step, off)
    k = end - start * jnp.int32(CHL)
    mvec = jnp.where(idx16 == rl, k, mvec)
    mvec = jnp.where(idx16 == RL + rl, start, mvec)
    ks.append(k)
    starts.append(start)
    off = lax.div(end + jnp.int32(CHL - 1), jnp.int32(CHL)) * jnp.int32(CHL)

  meta_vv[...] = mvec
  pltpu.sync_copy(vlist_v, vlist_hbm.at[c, s])
  pltpu.sync_copy(meta_vv, meta_hbm.at[c, s])

  for rl in range(RL):
    nch = lax.div(ks[rl] + jnp.int32(CHL - 1), jnp.int32(CHL))
    roff = jnp.int32(rl * NPAD)

    @pl.loop(0, nch)
    def _(j, start=starts[rl], roff=roff):
      base = (start + j) * CHL
      for q in range(CHL // 16):
        v = vlist_v[pl.ds(base + q * 16, 16)]
        dv = jnp.bitwise_and(lax.shift_right_logical(v, jnp.int32(3)),
                             jnp.int32(PACK - 1))
        irow[0, pl.ds(q * 16, 16)] = dv + roff
      pltpu.sync_copy(ones_v, counts_sp.at[irow.at[0]], add=True)

  plsc.subcore_barrier()

  for j in range(4):
    pltpu.sync_copy(counts_sp.at[pl.ds(s * CROWS_PT + j * 632, 632)], czero)
    pltpu.sync_copy(
        czero, counts_hbm.at[pl.ds(c * CROWS + s * CROWS_PT + j * 632, 632)])


_prep = pl.kernel(
    _prep_body,
    out_type=(
        jax.ShapeDtypeStruct((NC, NS, LV), jnp.int32),
        jax.ShapeDtypeStruct((NC, NS, 16), jnp.int32),
        jax.ShapeDtypeStruct((NC * CROWS, 16), jnp.float32),
    ),
    mesh=_mesh,
    scratch_types=[
        pltpu.VMEM_SHARED((CROWS, 16), jnp.float32),
        pltpu.VMEM((EPC,), jnp.int32),
        pltpu.VMEM((LV,), jnp.int32),
        pltpu.VMEM((16,), jnp.int32),
        pltpu.VMEM((CHL, 16), jnp.float32),
        pltpu.VMEM((1, CHL), jnp.int32),
        pltpu.VMEM((632, 16), jnp.float32),
    ],
    compiler_params=_sc_params,
)


def _agg_body(feats_hbm, vlist_hbm, meta_hbm, s_hbm,
              acc_sp, vbuf2, meta_vv, gbuf2, wbuf2, srow2, drow2,
              semL0, semL1, semG0, semG1, semS0, semS1, semZ, semW0, semW1):
  c = lax.axis_index("c")
  s = lax.axis_index("s")

  zb16 = jnp.zeros((32,), jnp.bfloat16)

  pltpu.sync_copy(meta_hbm.at[c, s], meta_vv)

  def unpack(p):
    for q in range(CHL // 16):
      v = vbuf2[p, pl.ds(q * 16, 16)]
      srow2[p, pl.ds(q * 16, 16)] = lax.shift_right_logical(v, jnp.int32(17))
      drow2[p, pl.ds(q * 16, 16)] = jnp.bitwise_and(
          lax.shift_right_logical(v, jnp.int32(3)), jnp.int32(PACK - 1))

  def fetch_list(chunk, p, sem):
    sems = (semL0, semL1)
    pltpu.async_copy(vlist_hbm.at[c, s, pl.ds(chunk * CHL, CHL)],
                     vbuf2.at[p], sems[p])

  for rl in range(RL):
    @pl.loop(0, CHL)
    def _(i):
      for q in range(D // 32):
        gbuf2[0, i, pl.ds(q * 32, 32)] = zb16

    zb = s * ACC_PT
    for t in range(NZC):
      pltpu.async_copy(gbuf2.at[0], acc_sp.at[pl.ds(zb + t * CHL, CHL)], semZ)
    pltpu.async_copy(gbuf2.at[0, pl.ds(0, ACC_TAIL)],
                     acc_sp.at[pl.ds(zb + NZC * CHL, ACC_TAIL)], semZ)
    for t in range(NZC):
      pltpu.make_async_copy(
          gbuf2.at[0], acc_sp.at[pl.ds(zb + t * CHL, CHL)], semZ).wait()
    pltpu.make_async_copy(gbuf2.at[0, pl.ds(0, ACC_TAIL)],
                          acc_sp.at[pl.ds(zb + NZC * CHL, ACC_TAIL)],
                          semZ).wait()
    plsc.subcore_barrier()

    mv = meta_vv[...]
    k = mv[rl]
    start = mv[RL + rl]
    nch = lax.div(k + jnp.int32(CHL - 1), jnp.int32(CHL))
    npair = lax.div(nch + jnp.int32(1), jnp.int32(2))

    @pl.when(nch > 0)
    def _():
      fetch_list(start, 0, semL0)

    @pl.loop(0, npair)
    def _(i):
      a = 2 * i
      b = 2 * i + 1

      @pl.when(b < nch)
      def _():
        fetch_list(start + b, 1, semL1)

      @pl.when(i > 0)
      def _():
        pltpu.make_async_copy(gbuf2.at[0], acc_sp.at[drow2.at[0]],
                              semS0).wait()
      pltpu.make_async_copy(vlist_hbm.at[c, s, pl.ds((start + a) * CHL, CHL)],
                            vbuf2.at[0], semL0).wait()
      unpack(0)
      pltpu.async_copy(feats_hbm.at[srow2.at[0]], gbuf2.at[0], semG0)

      @pl.when(b < nch)
      def _():
        @pl.when(a + 2 < nch)
        def _():
          fetch_list(start + a + 2, 0, semL0)

        @pl.when(i > 0)
        def _():
          pltpu.make_async_copy(gbuf2.at[1], acc_sp.at[drow2.at[1]],
                                semS1).wait()
        pltpu.make_async_copy(
            vlist_hbm.at[c, s, pl.ds((start + b) * CHL, CHL)],
            vbuf2.at[1], semL1).wait()
        unpack(1)
        pltpu.async_copy(feats_hbm.at[srow2.at[1]], gbuf2.at[1], semG1)

      pltpu.make_async_copy(feats_hbm.at[srow2.at[0]], gbuf2.at[0],
                            semG0).wait()
      pltpu.async_copy(gbuf2.at[0], acc_sp.at[drow2.at[0]], semS0, add=True)

      @pl.when(b < nch)
      def _():
        pltpu.make_async_copy(feats_hbm.at[srow2.at[1]], gbuf2.at[1],
                              semG1).wait()
        pltpu.async_copy(gbuf2.at[1], acc_sp.at[drow2.at[1]], semS1, add=True)

    @pl.when(nch > 0)
    def _():
      pltpu.make_async_copy(gbuf2.at[0], acc_sp.at[drow2.at[0]], semS0).wait()

    @pl.when(nch > 1)
    def _():
      pltpu.make_async_copy(gbuf2.at[1], acc_sp.at[drow2.at[1]], semS1).wait()

    plsc.subcore_barrier()

    r = c * RL + rl
    semW = (semW0, semW1)
    sizes = [CHL] * NZC + [ACC_TAIL]
    iota2 = lax.iota(jnp.int32, 16) * 2
    for t in range(NZC + 1):
      p = t % 2
      off = zb + t * CHL
      sz = sizes[t]
      if t >= 2:
        pltpu.make_async_copy(
            wbuf2.at[p, pl.ds(0, sizes[t - 2])],
            s_hbm.at[r, pl.ds(zb + (t - 2) * CHL, sizes[t - 2])],
            semW[p]).wait()
      pltpu.sync_copy(acc_sp.at[pl.ds(off, sz)], gbuf2.at[p, pl.ds(0, sz)])

      @pl.loop(0, sz)
      def _(i, p=p):
        for q in range(D // 32):
          v = gbuf2[p, i, pl.ds(q * 32, 32)]
          ev, od = plsc.unpack(v, format=plsc.PackFormat.INTERLEAVED)
          plsc.store_scatter(wbuf2.at[p, i], [iota2 + q * 32], ev)
          plsc.store_scatter(wbuf2.at[p, i], [iota2 + (q * 32 + 1)], od)

      pltpu.async_copy(wbuf2.at[p, pl.ds(0, sz)],
                       s_hbm.at[r, pl.ds(off, sz)], semW[p])
    pltpu.make_async_copy(wbuf2.at[1, pl.ds(0, sizes[NZC - 1])],
                          s_hbm.at[r, pl.ds(zb + (NZC - 1) * CHL,
                                            sizes[NZC - 1])],
                          semW1).wait()
    pltpu.make_async_copy(wbuf2.at[0, pl.ds(0, ACC_TAIL)],
                          s_hbm.at[r, pl.ds(zb + NZC * CHL, ACC_TAIL)],
                          semW0).wait()


_agg = pl.kernel(
    _agg_body,
    out_type=jax.ShapeDtypeStruct((R, NPAD, D), jnp.float32),
    mesh=_mesh,
    scratch_types=[
        pltpu.VMEM_SHARED((NPAD, D), jnp.bfloat16),
        pltpu.VMEM((2, CHL), jnp.int32),
        pltpu.VMEM((16,), jnp.int32),
        pltpu.VMEM((2, CHL, D), jnp.bfloat16),
        pltpu.VMEM((2, CHL, D), jnp.float32),
        pltpu.VMEM((2, CHL), jnp.int32),
        pltpu.VMEM((2, CHL), jnp.int32),
        pltpu.SemaphoreType.DMA,
        pltpu.SemaphoreType.DMA,
        pltpu.SemaphoreType.DMA,
        pltpu.SemaphoreType.DMA,
        pltpu.SemaphoreType.DMA,
        pltpu.SemaphoreType.DMA,
        pltpu.SemaphoreType.DMA,
        pltpu.SemaphoreType.DMA,
        pltpu.SemaphoreType.DMA,
    ],
    compiler_params=_sc_params,
)


BN = 400


def _tc_layer_body(x_ref, s_ref, cnt_ref, w_ref, root_ref, b_ref,
                   o_ref, ob_ref, *, relu):
  acc = jnp.dot(x_ref[...].astype(jnp.bfloat16), root_ref[...],
                preferred_element_type=jnp.float32)
  acc = acc + b_ref[...]
  for r in range(R):
    cnt = cnt_ref[r][:, 0:1]
    norm = 1.0 / jnp.maximum(cnt, 1.0)
    acc = acc + jnp.dot((s_ref[r] * norm).astype(jnp.bfloat16), w_ref[r],
                        preferred_element_type=jnp.float32)
  if relu:
    acc = jnp.maximum(acc, 0.0)
  o_ref[...] = acc
  ob_ref[...] = acc.astype(jnp.bfloat16)


def _tc_layer(feats, S, counts, W, root, b, relu):
  body = functools.partial(_tc_layer_body, relu=relu)
  return pl.pallas_call(
      body,
      grid=(N // BN,),
      in_specs=[
          pl.BlockSpec((BN, D), lambda i: (i, 0)),
          pl.BlockSpec((R, BN, D), lambda i: (0, i, 0)),
          pl.BlockSpec((R, BN, 16), lambda i: (0, i, 0)),
          pl.BlockSpec((R, D, D), lambda i: (0, 0, 0)),
          pl.BlockSpec((D, D), lambda i: (0, 0)),
          pl.BlockSpec((1, D), lambda i: (0, 0)),
      ],
      out_specs=[pl.BlockSpec((BN, D), lambda i: (i, 0)),
                 pl.BlockSpec((BN, D), lambda i: (i, 0))],
      out_shape=(jax.ShapeDtypeStruct((N, D), jnp.float32),
                 jax.ShapeDtypeStruct((N, D), jnp.bfloat16)),
  )(feats, S, counts, W, root, b)


def kernel(x, edge_index, edge_type, W1, root1, b1, W2, root2, b2):
  src = edge_index[0]
  dst = edge_index[1]
  packed = (src * PACK + dst) * 8 + edge_type
  vlist, meta, counts = _prep(packed)
  cnts = counts.reshape(R, NPAD, 16)
  w1b = W1.astype(jnp.bfloat16)
  w2b = W2.astype(jnp.bfloat16)
  r1b = root1.astype(jnp.bfloat16)
  r2b = root2.astype(jnp.bfloat16)
  S1 = _agg(x.astype(jnp.bfloat16), vlist, meta)
  h, hb = _tc_layer(x, S1, cnts, w1b, r1b, b1.reshape(1, D), True)
  S2 = _agg(hb, vlist, meta)
  out, _ = _tc_layer(h, S2, cnts, w2b, r2b, b2.reshape(1, D), False)
  return out
